# Initial kernel scaffold; baseline (speedup 1.0000x reference)
import jax
import jax.numpy as jnp
from jax.experimental import pallas as pl


def kernel(H, X_neis, V):
    raise NotImplementedError("write your pallas kernel here")


# speedup vs baseline: 7.9583x; 7.9583x over previous
import functools

import jax
import jax.numpy as jnp
from jax import lax
from jax.experimental import pallas as pl
from jax.experimental.pallas import tpu as pltpu
from jax.experimental.pallas import tpu_sc as plsc

E = 320000
D = 128
V_SEG = 10000
NC = 2
NS = 16
NW = NC * NS
RW = E // NW
BL = 80
NBL = RW // BL
NBUF = 4
VCHUNK = 1000
NVT = V_SEG // VCHUNK


def _sc_partial_segment_sum(H, ids3, zrows):
    mesh = plsc.VectorSubcoreMesh(
        core_axis_name="c", subcore_axis_name="s",
        num_cores=NC, num_subcores=NS)

    @functools.partial(
        pl.kernel,
        out_type=jax.ShapeDtypeStruct((NC, V_SEG, D), jnp.float32),
        mesh=mesh,
        scratch_types=[
            pltpu.VMEM((NBUF, BL, D), jnp.float32),
            pltpu.VMEM((NBUF, 1, BL), jnp.int32),
            pltpu.VMEM_SHARED((V_SEG, D), jnp.float32),
            [pltpu.SemaphoreType.DMA] * NBUF,
            [pltpu.SemaphoreType.DMA] * NBUF,
        ],
    )
    def k(h_hbm, ids_hbm, z_hbm, out_hbm, rows_v, ids_v, acc, lsems, ssems):
        c = lax.axis_index("c")
        s = lax.axis_index("s")
        wid = c * NS + s
        row_base = wid * RW
        vbase = s * VCHUNK

        @pl.when(s < NVT)
        def _zero():
            pltpu.sync_copy(z_hbm.at[pl.ds(vbase, VCHUNK), :],
                            acc.at[pl.ds(vbase, VCHUNK), :])

        plsc.subcore_barrier()

        def load_descs(blk, b):
            rows = pltpu.make_async_copy(
                h_hbm.at[pl.ds(row_base + blk * BL, BL), :],
                rows_v.at[b], lsems[b])
            ids = pltpu.make_async_copy(
                ids_hbm.at[wid, pl.ds(blk, 1), :], ids_v.at[b], lsems[b])
            return rows, ids

        def start_load(blk, b):
            for d in load_descs(blk, b):
                d.start()

        def wait_load(blk, b):
            for d in load_descs(blk, b):
                d.wait()

        def scatter_desc(b):
            return pltpu.make_async_copy(
                rows_v.at[b], acc.at[ids_v.at[b, 0]], ssems[b])

        for b in range(NBUF - 1):
            start_load(b, b)

        @pl.loop(0, NBL - 1, step=NBUF)
        def _ring(j):
            for b in range(NBUF):
                blk = j + b
                wait_load(blk, b)
                scatter_desc(b).start(add=True)
                nb = (b + NBUF - 1) % NBUF

                @pl.when(blk >= 1)
                def _drain():
                    scatter_desc(nb).wait()

                @pl.when(blk + NBUF - 1 < NBL)
                def _refill():
                    start_load(blk + NBUF - 1, nb)

        last = NBL - 1
        lb = last % NBUF
        wait_load(last, lb)
        scatter_desc(lb).start(add=True)
        scatter_desc((lb + NBUF - 1) % NBUF).wait()
        scatter_desc(lb).wait()
        plsc.subcore_barrier()

        @pl.when(s < NVT)
        def _write():
            pltpu.sync_copy(acc.at[pl.ds(vbase, VCHUNK), :],
                            out_hbm.at[c, pl.ds(vbase, VCHUNK), :])

    return k(H, ids3, zrows)


def _merge_partials(parts):
    BS = 1000

    def body(p_ref, o_ref):
        o_ref[...] = p_ref[0] + p_ref[1]

    return pl.pallas_call(
        body,
        grid=(V_SEG // BS,),
        in_specs=[pl.BlockSpec((NC, BS, D), lambda i: (0, i, 0))],
        out_specs=pl.BlockSpec((BS, D), lambda i: (i, 0)),
        out_shape=jax.ShapeDtypeStruct((V_SEG, D), jnp.float32),
    )(parts)


def kernel(H, X_neis, V):
    del V
    ids3 = X_neis.astype(jnp.int32).reshape(NW, NBL, BL)
    zrows = jnp.zeros((V_SEG, D), jnp.float32)
    parts = _sc_partial_segment_sum(H, ids3, zrows)
    return _merge_partials(parts)
